# baseline (speedup 1.0000x reference)
import functools

import jax
import jax.numpy as jnp
from jax import lax
from jax.experimental import pallas as pl
from jax.experimental.pallas import tpu as pltpu
from jax.experimental.pallas import tpu_sc as plsc

F32 = jnp.float32
I32 = jnp.int32

_NC, _NS = 2, 16
_NW = _NC * _NS
_CHUNK = 2000
_PREC = lax.Precision.HIGHEST


def _mm(a, w, b):
    return lax.dot_general(a, w, (((1,), (0,)), ((), ())),
                           preferred_element_type=F32, precision=_PREC) + b


def _relu(v):
    return jnp.maximum(v, 0.0)


def _deg_body(col_f, zeros_h, out_h, cb0, cb1, deg_v, sem0, sem1):
    cid = lax.axis_index("c")
    sid = lax.axis_index("s")
    n_pad = deg_v.shape[0]
    ept = col_f.shape[0] // _NW
    base = (cid * _NS + sid) * ept
    pltpu.sync_copy(zeros_h.at[pl.ds(0, n_pad)], deg_v)
    ones16 = jnp.full((16,), 1.0, F32)
    cbs, sems = [cb0, cb1], [sem0, sem1]
    nk = ept // _CHUNK

    def start(k):
        o = base + k * _CHUNK
        return pltpu.async_copy(col_f.at[pl.ds(o, _CHUNK)], cbs[k % 2], sems[k % 2])

    pend = start(0)
    for k in range(nk):
        pend.wait()
        if k + 1 < nk:
            nxt = start(k + 1)
        colb_v = cbs[k % 2]

        @plsc.parallel_loop(0, _CHUNK // 16, 1, unroll=5)
        def _(i):
            cv = colb_v[pl.ds(i * 16, 16)]
            plsc.addupdate_scatter(deg_v, [cv], ones16)

        if k + 1 < nk:
            pend = nxt
    pltpu.sync_copy(deg_v, out_h.at[cid, sid])


def _edge_body(g0_f, g1_f, row_f, col_f, zeros_h, out_h,
               rb0, cb0, rb1, cb1, g_v, acc_v, sem0, sem1):
    cid = lax.axis_index("c")
    sid = lax.axis_index("s")

    @pl.when(cid == 0)
    def _():
        pltpu.sync_copy(g0_f, g_v)

    @pl.when(cid == 1)
    def _():
        pltpu.sync_copy(g1_f, g_v)

    pltpu.sync_copy(zeros_h, acc_v)
    ept = row_f.shape[0] // _NS
    base = sid * ept
    rbs, cbs, sems = [rb0, rb1], [cb0, cb1], [sem0, sem1]
    nk = ept // _CHUNK

    def start(k):
        o = base + k * _CHUNK
        d1 = pltpu.async_copy(row_f.at[pl.ds(o, _CHUNK)], rbs[k % 2], sems[k % 2])
        d2 = pltpu.async_copy(col_f.at[pl.ds(o, _CHUNK)], cbs[k % 2], sems[k % 2])
        return (d1, d2)

    pend = start(0)
    for k in range(nk):
        pend[0].wait()
        pend[1].wait()
        if k + 1 < nk:
            nxt = start(k + 1)
        rowb_v, colb_v = rbs[k % 2], cbs[k % 2]

        @plsc.parallel_loop(0, _CHUNK // 16, 1, unroll=25)
        def _(i):
            rv = rowb_v[pl.ds(i * 16, 16)] * 4
            cv = colb_v[pl.ds(i * 16, 16)] * 4
            for fc in range(4):
                vals = plsc.load_gather(g_v, [cv + fc])
                plsc.addupdate_scatter(acc_v, [rv + fc], vals)

        if k + 1 < nk:
            pend = nxt
    pltpu.sync_copy(acc_v, out_h.at[cid, sid])


def _pre_body(x_ref, degs_ref, w1, b1, w2, b2, w3, b3,
              glo_ref, ghi_ref, dis_ref):
    deg = jnp.sum(degs_ref[...], axis=1)[:, None] + 1.0
    dis = lax.rsqrt(deg)
    h = _relu(_mm(x_ref[...], w1[...], b1[...]))
    h = _relu(_mm(h, w2[...], b2[...]))
    h = _mm(h, w3[...], b3[...])
    g = dis * h
    glo_ref[...] = g[:, :4]
    ghi_ref[...] = g[:, 4:]
    dis_ref[...] = dis


def _red_body(a_ref, lo_ref, hi_ref):
    a = a_ref[...]
    lo_ref[...] = jnp.sum(a[:_NS], axis=0)[None, :]
    hi_ref[...] = jnp.sum(a[_NS:], axis=0)[None, :]


def _post_body(a0_ref, a1_ref, glo_ref, ghi_ref, dis_ref, x_ref, batch_ref,
               mjob_ref,
               w1, b1, w2, b2, w3, b3,
               wd1, bd1, wd2, bd2, wd3, bd3,
               wg1, bg1, wg2, bg2, wg3, bg3,
               xn_ref, y_ref, z_ref, acc_y):
    i = pl.program_id(0)
    nsteps = pl.num_programs(0)
    acc8 = jnp.concatenate([a0_ref[...] + glo_ref[...],
                            a1_ref[...] + ghi_ref[...]], axis=1)
    aggr = dis_ref[...] * acc8
    t = _relu(_mm(aggr, w1[...], b1[...]))
    t = _relu(_mm(t, w2[...], b2[...]))
    xn = _mm(t, w3[...], b3[...])
    xn_ref[...] = xn
    ng = mjob_ref.shape[0]
    oh = (batch_ref[...] == lax.broadcasted_iota(I32, (batch_ref.shape[0], ng), 1)
          ).astype(F32)
    cat = jnp.concatenate([x_ref[...], xn], axis=1)
    part = lax.dot_general(oh, cat, (((0,), (0,)), ((), ())),
                           preferred_element_type=F32, precision=_PREC)

    @pl.when(i == 0)
    def _():
        acc_y[...] = jnp.zeros_like(acc_y)

    acc_y[...] += part

    @pl.when(i == nsteps - 1)
    def _():
        y = acc_y[...]
        t1 = _relu(_mm(y, wd1[...], bd1[...]))
        t1 = _relu(_mm(t1, wd2[...], bd2[...]))
        yd = _mm(t1, wd3[...], bd3[...])
        y_ref[...] = yd
        zp = lax.dot_general(mjob_ref[...], yd, (((1,), (0,)), ((), ())),
                             preferred_element_type=F32, precision=_PREC)
        t2 = _relu(_mm(zp, wg1[...], bg1[...]))
        t2 = _relu(_mm(t2, wg2[...], bg2[...]))
        z_ref[...] = _mm(t2, wg3[...], bg3[...])


def _full(shape):
    return pl.BlockSpec(shape, lambda i: tuple(0 for _ in shape))


def kernel(x, edge_index, batch, job_indptr, params):
    n = x.shape[0]
    e = edge_index.shape[1]
    in_ch = x.shape[1]
    ng = job_indptr.shape[0] - 1

    blk = 2000
    n_blocks = n // blk
    npw = n * 4

    row_f = edge_index[0]
    col_f = edge_index[1]
    batch2 = batch.reshape(n, 1)
    ids = jnp.arange(ng, dtype=I32)[None, :]
    mjob = ((ids >= job_indptr[:-1, None]) & (ids < job_indptr[1:, None])
            ).astype(F32)
    zeros_h = jnp.zeros((npw,), F32)

    mesh = plsc.VectorSubcoreMesh(core_axis_name="c", subcore_axis_name="s")
    sc_params = pltpu.CompilerParams(needs_layout_passes=False)

    deg_fn = pl.kernel(
        _deg_body,
        out_type=jax.ShapeDtypeStruct((_NC, _NS, n), F32),
        mesh=mesh,
        compiler_params=sc_params,
        scratch_types=[
            pltpu.VMEM((_CHUNK,), I32),
            pltpu.VMEM((_CHUNK,), I32),
            pltpu.VMEM((n,), F32),
            pltpu.SemaphoreType.DMA,
            pltpu.SemaphoreType.DMA,
        ],
    )
    deg_acc = deg_fn(col_f, zeros_h)
    degs = deg_acc.reshape(_NW, n).T

    p1 = params['mlp1']
    wb1 = []
    for wmat, bvec in p1:
        wb1 += [wmat, bvec.reshape(1, -1)]
    grid = (n_blocks,)
    pre = pl.pallas_call(
        _pre_body,
        grid=grid,
        in_specs=[
            pl.BlockSpec((blk, in_ch), lambda i: (i, 0)),
            pl.BlockSpec((blk, _NW), lambda i: (i, 0)),
        ] + [_full(a.shape) for a in wb1],
        out_specs=[
            pl.BlockSpec((blk, 4), lambda i: (i, 0)),
            pl.BlockSpec((blk, 4), lambda i: (i, 0)),
            pl.BlockSpec((blk, 1), lambda i: (i, 0)),
        ],
        out_shape=[
            jax.ShapeDtypeStruct((n, 4), F32),
            jax.ShapeDtypeStruct((n, 4), F32),
            jax.ShapeDtypeStruct((n, 1), F32),
        ],
    )
    g_lo, g_hi, dis = pre(x, degs, *wb1)
    g0f = g_lo.reshape(-1)
    g1f = g_hi.reshape(-1)

    edge_fn = pl.kernel(
        _edge_body,
        out_type=jax.ShapeDtypeStruct((_NC, _NS, npw), F32),
        mesh=mesh,
        compiler_params=sc_params,
        scratch_types=[
            pltpu.VMEM((_CHUNK,), I32),
            pltpu.VMEM((_CHUNK,), I32),
            pltpu.VMEM((_CHUNK,), I32),
            pltpu.VMEM((_CHUNK,), I32),
            pltpu.VMEM((npw,), F32),
            pltpu.VMEM((npw,), F32),
            pltpu.SemaphoreType.DMA,
            pltpu.SemaphoreType.DMA,
        ],
    )
    acc = edge_fn(g0f, g1f, row_f, col_f, zeros_h)
    accs2 = acc.reshape(_NW, npw)

    red = pl.pallas_call(
        _red_body,
        grid=(1,),
        in_specs=[pl.BlockSpec((_NW, npw), lambda i: (0, 0))],
        out_specs=[
            pl.BlockSpec((1, npw), lambda i: (0, 0)),
            pl.BlockSpec((1, npw), lambda i: (0, 0)),
        ],
        out_shape=[
            jax.ShapeDtypeStruct((1, npw), F32),
            jax.ShapeDtypeStruct((1, npw), F32),
        ],
    )
    rlo, rhi = red(accs2)
    a0 = rlo.reshape(n, 4)
    a1 = rhi.reshape(n, 4)

    wb = []
    for key in ('mlp2', 'mlp_dag', 'mlp_global'):
        for wmat, bvec in params[key]:
            wb += [wmat, bvec.reshape(1, -1)]
    post = pl.pallas_call(
        _post_body,
        grid=grid,
        in_specs=[
            pl.BlockSpec((blk, 4), lambda i: (i, 0)),
            pl.BlockSpec((blk, 4), lambda i: (i, 0)),
            pl.BlockSpec((blk, 4), lambda i: (i, 0)),
            pl.BlockSpec((blk, 4), lambda i: (i, 0)),
            pl.BlockSpec((blk, 1), lambda i: (i, 0)),
            pl.BlockSpec((blk, in_ch), lambda i: (i, 0)),
            pl.BlockSpec((blk, 1), lambda i: (i, 0)),
            _full(mjob.shape),
        ] + [_full(a.shape) for a in wb],
        out_specs=[
            pl.BlockSpec((blk, 128), lambda i: (i, 0)),
            _full((ng, 128)),
            _full((ng, 128)),
        ],
        out_shape=[
            jax.ShapeDtypeStruct((n, 128), F32),
            jax.ShapeDtypeStruct((ng, 128), F32),
            jax.ShapeDtypeStruct((ng, 128), F32),
        ],
        scratch_shapes=[pltpu.VMEM((ng, 256), F32)],
    )
    xn, y, z = post(a0, a1, g_lo, g_hi, dis, x, batch2, mjob, *wb)
    return (xn, y, z)


# speedup vs baseline: 1.0678x; 1.0678x over previous
import functools

import jax
import jax.numpy as jnp
from jax import lax
from jax.experimental import pallas as pl
from jax.experimental.pallas import tpu as pltpu
from jax.experimental.pallas import tpu_sc as plsc

F32 = jnp.float32
I32 = jnp.int32

_NC, _NS = 2, 16
_NW = _NC * _NS
_CHUNK = 2000
_PREC = lax.Precision.HIGHEST


def _mm(a, w, b):
    return lax.dot_general(a, w, (((1,), (0,)), ((), ())),
                           preferred_element_type=F32, precision=_PREC) + b


def _relu(v):
    return jnp.maximum(v, 0.0)


def _deg_body(col_f, zeros_h, out_h, cb0, cb1, deg_v, sem0, sem1):
    cid = lax.axis_index("c")
    sid = lax.axis_index("s")
    n_pad = deg_v.shape[0]
    ept = col_f.shape[0] // _NW
    base = (cid * _NS + sid) * ept
    pltpu.sync_copy(zeros_h.at[pl.ds(0, n_pad)], deg_v)
    ones16 = jnp.full((16,), 1.0, F32)
    cbs, sems = [cb0, cb1], [sem0, sem1]
    nk = ept // _CHUNK

    def start(k):
        o = base + k * _CHUNK
        return pltpu.async_copy(col_f.at[pl.ds(o, _CHUNK)], cbs[k % 2], sems[k % 2])

    pend = start(0)
    for k in range(nk):
        pend.wait()
        if k + 1 < nk:
            nxt = start(k + 1)
        colb_v = cbs[k % 2]

        @plsc.parallel_loop(0, _CHUNK // 16, 1, unroll=5)
        def _(i):
            cv = colb_v[pl.ds(i * 16, 16)]
            plsc.addupdate_scatter(deg_v, [cv], ones16)

        if k + 1 < nk:
            pend = nxt
    pltpu.sync_copy(deg_v, out_h.at[cid, sid])


def _edge_body(g0_f, g1_f, row_f, col_f, zeros_h, out_h,
               rb0, cb0, rb1, cb1, g_v, acc_v, sem0, sem1):
    cid = lax.axis_index("c")
    sid = lax.axis_index("s")

    @pl.when(cid == 0)
    def _():
        pltpu.sync_copy(g0_f, g_v)

    @pl.when(cid == 1)
    def _():
        pltpu.sync_copy(g1_f, g_v)

    pltpu.sync_copy(zeros_h, acc_v)
    ept = row_f.shape[0] // _NS
    base = sid * ept
    rbs, cbs, sems = [rb0, rb1], [cb0, cb1], [sem0, sem1]
    nk = ept // _CHUNK

    def start(k):
        o = base + k * _CHUNK
        d1 = pltpu.async_copy(row_f.at[pl.ds(o, _CHUNK)], rbs[k % 2], sems[k % 2])
        d2 = pltpu.async_copy(col_f.at[pl.ds(o, _CHUNK)], cbs[k % 2], sems[k % 2])
        return (d1, d2)

    pend = start(0)
    for k in range(nk):
        pend[0].wait()
        pend[1].wait()
        if k + 1 < nk:
            nxt = start(k + 1)
        rowb_v, colb_v = rbs[k % 2], cbs[k % 2]

        @plsc.parallel_loop(0, _CHUNK // 16, 1, unroll=5)
        def _(i):
            rv = rowb_v[pl.ds(i * 16, 16)] * 4
            cv = colb_v[pl.ds(i * 16, 16)] * 4
            for fc in range(4):
                vals = plsc.load_gather(g_v, [cv + fc])
                plsc.addupdate_scatter(acc_v, [rv + fc], vals)

        if k + 1 < nk:
            pend = nxt
    pltpu.sync_copy(acc_v, out_h.at[cid, sid])


def _pre_body(x_ref, degs_ref, w1, b1, w2, b2, w3, b3,
              glo_ref, ghi_ref, dis_ref):
    deg = jnp.sum(degs_ref[...], axis=1)[:, None] + 1.0
    dis = lax.rsqrt(deg)
    h = _relu(_mm(x_ref[...], w1[...], b1[...]))
    h = _relu(_mm(h, w2[...], b2[...]))
    h = _mm(h, w3[...], b3[...])
    g = dis * h
    glo_ref[...] = g[:, :4]
    ghi_ref[...] = g[:, 4:]
    dis_ref[...] = dis


def _red_body(a_ref, lo_ref, hi_ref):
    a = a_ref[...]
    lo_ref[...] = jnp.sum(a[:_NS], axis=0)[None, :]
    hi_ref[...] = jnp.sum(a[_NS:], axis=0)[None, :]


def _post_body(a0_ref, a1_ref, glo_ref, ghi_ref, dis_ref, x_ref, batch_ref,
               mjob_ref,
               w1, b1, w2, b2, w3, b3,
               wd1, bd1, wd2, bd2, wd3, bd3,
               wg1, bg1, wg2, bg2, wg3, bg3,
               xn_ref, y_ref, z_ref, acc_y):
    i = pl.program_id(0)
    nsteps = pl.num_programs(0)
    acc8 = jnp.concatenate([a0_ref[...] + glo_ref[...],
                            a1_ref[...] + ghi_ref[...]], axis=1)
    aggr = dis_ref[...] * acc8
    t = _relu(_mm(aggr, w1[...], b1[...]))
    t = _relu(_mm(t, w2[...], b2[...]))
    xn = _mm(t, w3[...], b3[...])
    xn_ref[...] = xn
    ng = mjob_ref.shape[0]
    oh = (batch_ref[...] == lax.broadcasted_iota(I32, (batch_ref.shape[0], ng), 1)
          ).astype(F32)
    cat = jnp.concatenate([x_ref[...], xn], axis=1)
    part = lax.dot_general(oh, cat, (((0,), (0,)), ((), ())),
                           preferred_element_type=F32, precision=_PREC)

    @pl.when(i == 0)
    def _():
        acc_y[...] = jnp.zeros_like(acc_y)

    acc_y[...] += part

    @pl.when(i == nsteps - 1)
    def _():
        y = acc_y[...]
        t1 = _relu(_mm(y, wd1[...], bd1[...]))
        t1 = _relu(_mm(t1, wd2[...], bd2[...]))
        yd = _mm(t1, wd3[...], bd3[...])
        y_ref[...] = yd
        zp = lax.dot_general(mjob_ref[...], yd, (((1,), (0,)), ((), ())),
                             preferred_element_type=F32, precision=_PREC)
        t2 = _relu(_mm(zp, wg1[...], bg1[...]))
        t2 = _relu(_mm(t2, wg2[...], bg2[...]))
        z_ref[...] = _mm(t2, wg3[...], bg3[...])


def _full(shape):
    return pl.BlockSpec(shape, lambda i: tuple(0 for _ in shape))


def kernel(x, edge_index, batch, job_indptr, params):
    n = x.shape[0]
    e = edge_index.shape[1]
    in_ch = x.shape[1]
    ng = job_indptr.shape[0] - 1

    blk = 2000
    n_blocks = n // blk
    npw = n * 4

    row_f = edge_index[0]
    col_f = edge_index[1]
    batch2 = batch.reshape(n, 1)
    ids = jnp.arange(ng, dtype=I32)[None, :]
    mjob = ((ids >= job_indptr[:-1, None]) & (ids < job_indptr[1:, None])
            ).astype(F32)
    zeros_h = jnp.zeros((npw,), F32)

    mesh = plsc.VectorSubcoreMesh(core_axis_name="c", subcore_axis_name="s")
    sc_params = pltpu.CompilerParams(needs_layout_passes=False)

    deg_fn = pl.kernel(
        _deg_body,
        out_type=jax.ShapeDtypeStruct((_NC, _NS, n), F32),
        mesh=mesh,
        compiler_params=sc_params,
        scratch_types=[
            pltpu.VMEM((_CHUNK,), I32),
            pltpu.VMEM((_CHUNK,), I32),
            pltpu.VMEM((n,), F32),
            pltpu.SemaphoreType.DMA,
            pltpu.SemaphoreType.DMA,
        ],
    )
    deg_acc = deg_fn(col_f, zeros_h)
    degs = deg_acc.reshape(_NW, n).T

    p1 = params['mlp1']
    wb1 = []
    for wmat, bvec in p1:
        wb1 += [wmat, bvec.reshape(1, -1)]
    grid = (n_blocks,)
    pre = pl.pallas_call(
        _pre_body,
        grid=grid,
        in_specs=[
            pl.BlockSpec((blk, in_ch), lambda i: (i, 0)),
            pl.BlockSpec((blk, _NW), lambda i: (i, 0)),
        ] + [_full(a.shape) for a in wb1],
        out_specs=[
            pl.BlockSpec((blk, 4), lambda i: (i, 0)),
            pl.BlockSpec((blk, 4), lambda i: (i, 0)),
            pl.BlockSpec((blk, 1), lambda i: (i, 0)),
        ],
        out_shape=[
            jax.ShapeDtypeStruct((n, 4), F32),
            jax.ShapeDtypeStruct((n, 4), F32),
            jax.ShapeDtypeStruct((n, 1), F32),
        ],
    )
    g_lo, g_hi, dis = pre(x, degs, *wb1)
    g0f = g_lo.reshape(-1)
    g1f = g_hi.reshape(-1)

    edge_fn = pl.kernel(
        _edge_body,
        out_type=jax.ShapeDtypeStruct((_NC, _NS, npw), F32),
        mesh=mesh,
        compiler_params=sc_params,
        scratch_types=[
            pltpu.VMEM((_CHUNK,), I32),
            pltpu.VMEM((_CHUNK,), I32),
            pltpu.VMEM((_CHUNK,), I32),
            pltpu.VMEM((_CHUNK,), I32),
            pltpu.VMEM((npw,), F32),
            pltpu.VMEM((npw,), F32),
            pltpu.SemaphoreType.DMA,
            pltpu.SemaphoreType.DMA,
        ],
    )
    acc = edge_fn(g0f, g1f, row_f, col_f, zeros_h)
    accs2 = acc.reshape(_NW, npw)

    red = pl.pallas_call(
        _red_body,
        grid=(1,),
        in_specs=[pl.BlockSpec((_NW, npw), lambda i: (0, 0))],
        out_specs=[
            pl.BlockSpec((1, npw), lambda i: (0, 0)),
            pl.BlockSpec((1, npw), lambda i: (0, 0)),
        ],
        out_shape=[
            jax.ShapeDtypeStruct((1, npw), F32),
            jax.ShapeDtypeStruct((1, npw), F32),
        ],
    )
    rlo, rhi = red(accs2)
    a0 = rlo.reshape(n, 4)
    a1 = rhi.reshape(n, 4)

    wb = []
    for key in ('mlp2', 'mlp_dag', 'mlp_global'):
        for wmat, bvec in params[key]:
            wb += [wmat, bvec.reshape(1, -1)]
    post = pl.pallas_call(
        _post_body,
        grid=grid,
        in_specs=[
            pl.BlockSpec((blk, 4), lambda i: (i, 0)),
            pl.BlockSpec((blk, 4), lambda i: (i, 0)),
            pl.BlockSpec((blk, 4), lambda i: (i, 0)),
            pl.BlockSpec((blk, 4), lambda i: (i, 0)),
            pl.BlockSpec((blk, 1), lambda i: (i, 0)),
            pl.BlockSpec((blk, in_ch), lambda i: (i, 0)),
            pl.BlockSpec((blk, 1), lambda i: (i, 0)),
            _full(mjob.shape),
        ] + [_full(a.shape) for a in wb],
        out_specs=[
            pl.BlockSpec((blk, 128), lambda i: (i, 0)),
            _full((ng, 128)),
            _full((ng, 128)),
        ],
        out_shape=[
            jax.ShapeDtypeStruct((n, 128), F32),
            jax.ShapeDtypeStruct((ng, 128), F32),
            jax.ShapeDtypeStruct((ng, 128), F32),
        ],
        scratch_shapes=[pltpu.VMEM((ng, 256), F32)],
    )
    xn, y, z = post(a0, a1, g_lo, g_hi, dis, x, batch2, mjob, *wb)
    return (xn, y, z)


# speedup vs baseline: 1.0709x; 1.0029x over previous
import functools

import jax
import jax.numpy as jnp
from jax import lax
from jax.experimental import pallas as pl
from jax.experimental.pallas import tpu as pltpu
from jax.experimental.pallas import tpu_sc as plsc

F32 = jnp.float32
I32 = jnp.int32

_NC, _NS = 2, 16
_NW = _NC * _NS
_CHUNK = 2000
_ECHUNK = 4000
_PREC = lax.Precision.HIGHEST


def _mm(a, w, b):
    return lax.dot_general(a, w, (((1,), (0,)), ((), ())),
                           preferred_element_type=F32, precision=_PREC) + b


def _relu(v):
    return jnp.maximum(v, 0.0)


def _deg_body(col_f, zeros_h, out_h, cb0, cb1, deg_v, sem0, sem1):
    cid = lax.axis_index("c")
    sid = lax.axis_index("s")
    n_pad = deg_v.shape[0]
    ept = col_f.shape[0] // _NW
    base = (cid * _NS + sid) * ept
    pltpu.sync_copy(zeros_h.at[pl.ds(0, n_pad)], deg_v)
    ones16 = jnp.full((16,), 1.0, F32)
    cbs, sems = [cb0, cb1], [sem0, sem1]
    nk = ept // _CHUNK

    def start(k):
        o = base + k * _CHUNK
        return pltpu.async_copy(col_f.at[pl.ds(o, _CHUNK)], cbs[k % 2], sems[k % 2])

    pend = start(0)
    for k in range(nk):
        pend.wait()
        if k + 1 < nk:
            nxt = start(k + 1)
        colb_v = cbs[k % 2]

        @plsc.parallel_loop(0, _CHUNK // 16, 1, unroll=5)
        def _(i):
            cv = colb_v[pl.ds(i * 16, 16)]
            plsc.addupdate_scatter(deg_v, [cv], ones16)

        if k + 1 < nk:
            pend = nxt
    pltpu.sync_copy(deg_v, out_h.at[cid, sid])


def _edge_body(g0_f, g1_f, row_f, col_f, zeros_h, out_h,
               rb0, cb0, rb1, cb1, g_v, acc_v, sem0, sem1):
    cid = lax.axis_index("c")
    sid = lax.axis_index("s")

    @pl.when(cid == 0)
    def _():
        pltpu.sync_copy(g0_f, g_v)

    @pl.when(cid == 1)
    def _():
        pltpu.sync_copy(g1_f, g_v)

    pltpu.sync_copy(zeros_h, acc_v)
    ept = row_f.shape[0] // _NS
    base = sid * ept
    rbs, cbs, sems = [rb0, rb1], [cb0, cb1], [sem0, sem1]
    nk = ept // _ECHUNK

    def start(k):
        o = base + k * _ECHUNK
        d1 = pltpu.async_copy(row_f.at[pl.ds(o, _ECHUNK)], rbs[k % 2], sems[k % 2])
        d2 = pltpu.async_copy(col_f.at[pl.ds(o, _ECHUNK)], cbs[k % 2], sems[k % 2])
        return (d1, d2)

    pend = start(0)
    for k in range(nk):
        pend[0].wait()
        pend[1].wait()
        if k + 1 < nk:
            nxt = start(k + 1)
        rowb_v, colb_v = rbs[k % 2], cbs[k % 2]

        @plsc.parallel_loop(0, _ECHUNK // 16, 1, unroll=5)
        def _(i):
            rv = rowb_v[pl.ds(i * 16, 16)] * 4
            cv = colb_v[pl.ds(i * 16, 16)] * 4
            for fc in range(4):
                vals = plsc.load_gather(g_v, [cv + fc])
                plsc.addupdate_scatter(acc_v, [rv + fc], vals)

        if k + 1 < nk:
            pend = nxt
    pltpu.sync_copy(acc_v, out_h.at[cid, sid])


def _pre_body(x_ref, degs_ref, w1, b1, w2, b2, w3, b3,
              glo_ref, ghi_ref, dis_ref):
    deg = jnp.sum(degs_ref[...], axis=1)[:, None] + 1.0
    dis = lax.rsqrt(deg)
    h = _relu(_mm(x_ref[...], w1[...], b1[...]))
    h = _relu(_mm(h, w2[...], b2[...]))
    h = _mm(h, w3[...], b3[...])
    g = dis * h
    glo_ref[...] = g[:, :4]
    ghi_ref[...] = g[:, 4:]
    dis_ref[...] = dis


def _red_body(a_ref, lo_ref, hi_ref):
    a = a_ref[...]
    lo_ref[...] = jnp.sum(a[:_NS], axis=0)[None, :]
    hi_ref[...] = jnp.sum(a[_NS:], axis=0)[None, :]


def _post_body(a0_ref, a1_ref, glo_ref, ghi_ref, dis_ref, x_ref, batch_ref,
               mjob_ref,
               w1, b1, w2, b2, w3, b3,
               wd1, bd1, wd2, bd2, wd3, bd3,
               wg1, bg1, wg2, bg2, wg3, bg3,
               xn_ref, y_ref, z_ref, acc_y):
    i = pl.program_id(0)
    nsteps = pl.num_programs(0)
    acc8 = jnp.concatenate([a0_ref[...] + glo_ref[...],
                            a1_ref[...] + ghi_ref[...]], axis=1)
    aggr = dis_ref[...] * acc8
    t = _relu(_mm(aggr, w1[...], b1[...]))
    t = _relu(_mm(t, w2[...], b2[...]))
    xn = _mm(t, w3[...], b3[...])
    xn_ref[...] = xn
    ng = mjob_ref.shape[0]
    oh = (batch_ref[...] == lax.broadcasted_iota(I32, (batch_ref.shape[0], ng), 1)
          ).astype(F32)
    cat = jnp.concatenate([x_ref[...], xn], axis=1)
    part = lax.dot_general(oh, cat, (((0,), (0,)), ((), ())),
                           preferred_element_type=F32, precision=_PREC)

    @pl.when(i == 0)
    def _():
        acc_y[...] = jnp.zeros_like(acc_y)

    acc_y[...] += part

    @pl.when(i == nsteps - 1)
    def _():
        y = acc_y[...]
        t1 = _relu(_mm(y, wd1[...], bd1[...]))
        t1 = _relu(_mm(t1, wd2[...], bd2[...]))
        yd = _mm(t1, wd3[...], bd3[...])
        y_ref[...] = yd
        zp = lax.dot_general(mjob_ref[...], yd, (((1,), (0,)), ((), ())),
                             preferred_element_type=F32, precision=_PREC)
        t2 = _relu(_mm(zp, wg1[...], bg1[...]))
        t2 = _relu(_mm(t2, wg2[...], bg2[...]))
        z_ref[...] = _mm(t2, wg3[...], bg3[...])


def _full(shape):
    return pl.BlockSpec(shape, lambda i: tuple(0 for _ in shape))


def kernel(x, edge_index, batch, job_indptr, params):
    n = x.shape[0]
    e = edge_index.shape[1]
    in_ch = x.shape[1]
    ng = job_indptr.shape[0] - 1

    blk = 2000
    n_blocks = n // blk
    npw = n * 4

    row_f = edge_index[0]
    col_f = edge_index[1]
    batch2 = batch.reshape(n, 1)
    ids = jnp.arange(ng, dtype=I32)[None, :]
    mjob = ((ids >= job_indptr[:-1, None]) & (ids < job_indptr[1:, None])
            ).astype(F32)
    zeros_h = jnp.zeros((npw,), F32)

    mesh = plsc.VectorSubcoreMesh(core_axis_name="c", subcore_axis_name="s")
    sc_params = pltpu.CompilerParams(needs_layout_passes=False)

    deg_fn = pl.kernel(
        _deg_body,
        out_type=jax.ShapeDtypeStruct((_NC, _NS, n), F32),
        mesh=mesh,
        compiler_params=sc_params,
        scratch_types=[
            pltpu.VMEM((_CHUNK,), I32),
            pltpu.VMEM((_CHUNK,), I32),
            pltpu.VMEM((n,), F32),
            pltpu.SemaphoreType.DMA,
            pltpu.SemaphoreType.DMA,
        ],
    )
    deg_acc = deg_fn(col_f, zeros_h)
    degs = deg_acc.reshape(_NW, n).T

    p1 = params['mlp1']
    wb1 = []
    for wmat, bvec in p1:
        wb1 += [wmat, bvec.reshape(1, -1)]
    grid = (n_blocks,)
    pre = pl.pallas_call(
        _pre_body,
        grid=grid,
        in_specs=[
            pl.BlockSpec((blk, in_ch), lambda i: (i, 0)),
            pl.BlockSpec((blk, _NW), lambda i: (i, 0)),
        ] + [_full(a.shape) for a in wb1],
        out_specs=[
            pl.BlockSpec((blk, 4), lambda i: (i, 0)),
            pl.BlockSpec((blk, 4), lambda i: (i, 0)),
            pl.BlockSpec((blk, 1), lambda i: (i, 0)),
        ],
        out_shape=[
            jax.ShapeDtypeStruct((n, 4), F32),
            jax.ShapeDtypeStruct((n, 4), F32),
            jax.ShapeDtypeStruct((n, 1), F32),
        ],
    )
    g_lo, g_hi, dis = pre(x, degs, *wb1)
    g0f = g_lo.reshape(-1)
    g1f = g_hi.reshape(-1)

    edge_fn = pl.kernel(
        _edge_body,
        out_type=jax.ShapeDtypeStruct((_NC, _NS, npw), F32),
        mesh=mesh,
        compiler_params=sc_params,
        scratch_types=[
            pltpu.VMEM((_ECHUNK,), I32),
            pltpu.VMEM((_ECHUNK,), I32),
            pltpu.VMEM((_ECHUNK,), I32),
            pltpu.VMEM((_ECHUNK,), I32),
            pltpu.VMEM((npw,), F32),
            pltpu.VMEM((npw,), F32),
            pltpu.SemaphoreType.DMA,
            pltpu.SemaphoreType.DMA,
        ],
    )
    acc = edge_fn(g0f, g1f, row_f, col_f, zeros_h)
    accs2 = acc.reshape(_NW, npw)

    red = pl.pallas_call(
        _red_body,
        grid=(1,),
        in_specs=[pl.BlockSpec((_NW, npw), lambda i: (0, 0))],
        out_specs=[
            pl.BlockSpec((1, npw), lambda i: (0, 0)),
            pl.BlockSpec((1, npw), lambda i: (0, 0)),
        ],
        out_shape=[
            jax.ShapeDtypeStruct((1, npw), F32),
            jax.ShapeDtypeStruct((1, npw), F32),
        ],
    )
    rlo, rhi = red(accs2)
    a0 = rlo.reshape(n, 4)
    a1 = rhi.reshape(n, 4)

    wb = []
    for key in ('mlp2', 'mlp_dag', 'mlp_global'):
        for wmat, bvec in params[key]:
            wb += [wmat, bvec.reshape(1, -1)]
    post = pl.pallas_call(
        _post_body,
        grid=grid,
        in_specs=[
            pl.BlockSpec((blk, 4), lambda i: (i, 0)),
            pl.BlockSpec((blk, 4), lambda i: (i, 0)),
            pl.BlockSpec((blk, 4), lambda i: (i, 0)),
            pl.BlockSpec((blk, 4), lambda i: (i, 0)),
            pl.BlockSpec((blk, 1), lambda i: (i, 0)),
            pl.BlockSpec((blk, in_ch), lambda i: (i, 0)),
            pl.BlockSpec((blk, 1), lambda i: (i, 0)),
            _full(mjob.shape),
        ] + [_full(a.shape) for a in wb],
        out_specs=[
            pl.BlockSpec((blk, 128), lambda i: (i, 0)),
            _full((ng, 128)),
            _full((ng, 128)),
        ],
        out_shape=[
            jax.ShapeDtypeStruct((n, 128), F32),
            jax.ShapeDtypeStruct((ng, 128), F32),
            jax.ShapeDtypeStruct((ng, 128), F32),
        ],
        scratch_shapes=[pltpu.VMEM((ng, 256), F32)],
    )
    xn, y, z = post(a0, a1, g_lo, g_hi, dis, x, batch2, mjob, *wb)
    return (xn, y, z)
